# Initial kernel scaffold; baseline (speedup 1.0000x reference)
import jax
import jax.numpy as jnp
from jax.experimental import pallas as pl


def kernel(ground_truth_mask, ground_truth_contour, snake_GT_size, snake_classic_size, snake_mask, classic_contour, classic_mask):
    raise NotImplementedError("write your pallas kernel here")


# speedup vs baseline: 23.1469x; 23.1469x over previous
import jax
import jax.numpy as jnp
from jax.experimental import pallas as pl
from jax.experimental.pallas import tpu as pltpu

_GAMMA = 0.5
_SMOOTH = 1.0
_B = 64
_N = 512
_W = 512
_STEPS = 32


def _mask_kernel(c_ref, g_ref, s_ref, out_ref):
    j = pl.program_id(1)
    c = c_ref[...]
    g = g_ref[...]
    s = s_ref[...]
    sc = jnp.sum(c, axis=0, keepdims=True)
    sg = jnp.sum(g, axis=0, keepdims=True)
    ss = jnp.sum(s, axis=0, keepdims=True)
    scg = jnp.sum(c * g, axis=0, keepdims=True)
    scs = jnp.sum(c * s, axis=0, keepdims=True)
    block = jnp.concatenate(
        [sc, sg, ss, scg, scs, jnp.zeros((3, _W), jnp.float32)], axis=0)[None]

    @pl.when(j == 0)
    def _():
        out_ref[...] = block

    @pl.when(j != 0)
    def _():
        out_ref[...] = out_ref[...] + block


def _corr_kernel(pbase_ref, marr_ref, part_ref, out_ref, pfull_ref, acc_ref):
    p0 = pbase_ref[...]
    pfull_ref[0:256, :] = p0
    pfull_ref[256:512, :] = pltpu.roll(p0, 384, axis=1)
    pfull_ref[512:768, :] = pltpu.roll(p0, 256, axis=1)
    pfull_ref[768:1024, :] = pltpu.roll(p0, 128, axis=1)

    def body(w, carry):
        pr = pltpu.roll(pfull_ref[...], (_N - w) % _N, axis=1)
        m = pltpu.roll(marr_ref[...], (128 - w) % 128, axis=1)[:, 0:1]
        t = (m * pr).reshape(4, 256, _N)
        contrib = t[0] + t[1] + t[2] + t[3]

        @pl.when(w == 0)
        def _():
            acc_ref[...] = contrib

        @pl.when(w != 0)
        def _():
            acc_ref[...] = acc_ref[...] + contrib

        return carry

    jax.lax.fori_loop(0, 128, body, 0)

    acc = acc_ref[...]
    corr_q0 = acc[0:64, :] + acc[64:128, :]
    corr_q1 = acc[128:192, :] + acc[192:256, :]
    cmax0 = jnp.max(corr_q0, axis=1, keepdims=True)
    cmax1 = jnp.max(corr_q1, axis=1, keepdims=True)

    prow = jnp.sum(p0 * p0, axis=1, keepdims=True)
    ap0 = prow[0:64] + prow[64:128]
    ap1 = prow[128:192] + prow[192:256]
    mar = marr_ref[...]
    mm = jnp.sum(mar * mar, axis=1, keepdims=True)
    mu = mm[0:256] + mm[256:512] + mm[512:768] + mm[768:1024]
    ar0 = mu[0:64] + mu[64:128]
    ar1 = mu[128:192] + mu[192:256]

    inv2n = 1.0 / (2.0 * _N)
    mse0 = (ap0 + ar0 - 2.0 * cmax0) * inv2n
    mse1 = (ap1 + ar1 - 2.0 * cmax1) * inv2n
    seg = jnp.sum(mse0) * (1.0 / _B)
    cons = jnp.sum(mse1) * (1.0 / _B)

    p5 = part_ref[0] + part_ref[1]
    s_c = jnp.sum(p5[0:1, :])
    s_g = jnp.sum(p5[1:2, :])
    s_s = jnp.sum(p5[2:3, :])
    s_cg = jnp.sum(p5[3:4, :])
    s_cs = jnp.sum(p5[4:5, :])
    dice1 = 1.0 - (2.0 * s_cg + _SMOOTH) / (s_c + s_g + _SMOOTH)
    dice2 = 1.0 - (2.0 * s_cs + _SMOOTH) / (s_c + s_s + _SMOOTH)
    loss = (1.0 - _GAMMA) * (dice1 + seg) + _GAMMA * (dice2 + cons)
    out_ref[...] = jnp.reshape(loss, (1, 1))


def kernel(ground_truth_mask, ground_truth_contour, snake_GT_size,
           snake_classic_size, snake_mask, classic_contour, classic_mask):
    B, N, W = _B, _N, _W
    c2 = classic_mask.reshape(B * 512, W)
    g2 = ground_truth_mask.reshape(B * 512, W)
    s2 = snake_mask.reshape(B * 512, W)
    rows = (B * 512) // (2 * _STEPS)

    part = pl.pallas_call(
        _mask_kernel,
        grid=(2, _STEPS),
        in_specs=[
            pl.BlockSpec((rows, W), lambda i, j: (i * _STEPS + j, 0)),
            pl.BlockSpec((rows, W), lambda i, j: (i * _STEPS + j, 0)),
            pl.BlockSpec((rows, W), lambda i, j: (i * _STEPS + j, 0)),
        ],
        out_specs=pl.BlockSpec((1, 8, W), lambda i, j: (i, 0, 0)),
        out_shape=jax.ShapeDtypeStruct((2, 8, W), jnp.float32),
        compiler_params=pltpu.CompilerParams(
            dimension_semantics=("parallel", "arbitrary")),
    )(c2, g2, s2)

    preds = jnp.stack([snake_GT_size, snake_classic_size])
    refs = jnp.stack([ground_truth_contour, classic_contour])
    pbase = preds.transpose(0, 3, 1, 2).reshape(2 * 2 * B, N)
    marr = (refs.transpose(0, 3, 1, 2)
            .reshape(2, 2, B, 4, 128)
            .transpose(3, 0, 1, 2, 4)
            .reshape(4 * 2 * 2 * B, 128))

    out = pl.pallas_call(
        _corr_kernel,
        in_specs=[
            pl.BlockSpec((2 * 2 * B, N), lambda: (0, 0)),
            pl.BlockSpec((4 * 2 * 2 * B, 128), lambda: (0, 0)),
            pl.BlockSpec((2, 8, W), lambda: (0, 0, 0)),
        ],
        out_specs=pl.BlockSpec((1, 1), lambda: (0, 0)),
        out_shape=jax.ShapeDtypeStruct((1, 1), jnp.float32),
        scratch_shapes=[
            pltpu.VMEM((4 * 2 * 2 * B, N), jnp.float32),
            pltpu.VMEM((2 * 2 * B, N), jnp.float32),
        ],
    )(pbase, marr, part)
    return out[0, 0]


# speedup vs baseline: 33.7207x; 1.4568x over previous
import jax
import jax.numpy as jnp
from jax.experimental import pallas as pl
from jax.experimental.pallas import tpu as pltpu

_GAMMA = 0.5
_SMOOTH = 1.0
_B = 64
_N = 512
_W = 512
_STEPS = 32


def _mask_kernel(c_ref, g_ref, s_ref, out_ref):
    j = pl.program_id(1)
    c = c_ref[...]
    g = g_ref[...]
    s = s_ref[...]
    sc = jnp.sum(c, axis=0, keepdims=True)
    sg = jnp.sum(g, axis=0, keepdims=True)
    ss = jnp.sum(s, axis=0, keepdims=True)
    scg = jnp.sum(c * g, axis=0, keepdims=True)
    scs = jnp.sum(c * s, axis=0, keepdims=True)
    block = jnp.concatenate(
        [sc, sg, ss, scg, scs, jnp.zeros((3, _W), jnp.float32)], axis=0)[None]

    @pl.when(j == 0)
    def _():
        out_ref[...] = block

    @pl.when(j != 0)
    def _():
        out_ref[...] = out_ref[...] + block


def _corr_kernel(pbase_ref, marr_ref, part_ref, out_ref, pfull_ref, acc_ref,
                 mcur_ref):
    p0 = pbase_ref[...]
    pfull_ref[0:256, :] = p0
    pfull_ref[256:512, :] = pltpu.roll(p0, 384, axis=1)
    pfull_ref[512:768, :] = pltpu.roll(p0, 256, axis=1)
    pfull_ref[768:1024, :] = pltpu.roll(p0, 128, axis=1)
    mcur_ref[...] = pltpu.roll(marr_ref[...], 1, axis=1)
    acc_ref[...] = jnp.zeros((256, _N), jnp.float32)

    def body(it, carry):
        m = mcur_ref[:, 0:1]
        t = (m * pfull_ref[...]).reshape(4, 256, _N)
        v = t[0] + t[1] + t[2] + t[3]
        acc_ref[...] = pltpu.roll(acc_ref[...], _N - 1, axis=1) + v
        mcur_ref[...] = pltpu.roll(mcur_ref[...], 1, axis=1)
        return carry

    jax.lax.fori_loop(0, 128, body, 0)

    acc = acc_ref[...]
    corr_q0 = acc[0:64, :] + acc[64:128, :]
    corr_q1 = acc[128:192, :] + acc[192:256, :]
    cmax0 = jnp.max(corr_q0, axis=1, keepdims=True)
    cmax1 = jnp.max(corr_q1, axis=1, keepdims=True)

    prow = jnp.sum(p0 * p0, axis=1, keepdims=True)
    ap0 = prow[0:64] + prow[64:128]
    ap1 = prow[128:192] + prow[192:256]
    mar = marr_ref[...]
    mm = jnp.sum(mar * mar, axis=1, keepdims=True)
    mu = mm[0:256] + mm[256:512] + mm[512:768] + mm[768:1024]
    ar0 = mu[0:64] + mu[64:128]
    ar1 = mu[128:192] + mu[192:256]

    inv2n = 1.0 / (2.0 * _N)
    mse0 = (ap0 + ar0 - 2.0 * cmax0) * inv2n
    mse1 = (ap1 + ar1 - 2.0 * cmax1) * inv2n
    seg = jnp.sum(mse0) * (1.0 / _B)
    cons = jnp.sum(mse1) * (1.0 / _B)

    p5 = part_ref[0] + part_ref[1]
    s_c = jnp.sum(p5[0:1, :])
    s_g = jnp.sum(p5[1:2, :])
    s_s = jnp.sum(p5[2:3, :])
    s_cg = jnp.sum(p5[3:4, :])
    s_cs = jnp.sum(p5[4:5, :])
    dice1 = 1.0 - (2.0 * s_cg + _SMOOTH) / (s_c + s_g + _SMOOTH)
    dice2 = 1.0 - (2.0 * s_cs + _SMOOTH) / (s_c + s_s + _SMOOTH)
    loss = (1.0 - _GAMMA) * (dice1 + seg) + _GAMMA * (dice2 + cons)
    out_ref[...] = jnp.reshape(loss, (1, 1))


def kernel(ground_truth_mask, ground_truth_contour, snake_GT_size,
           snake_classic_size, snake_mask, classic_contour, classic_mask):
    B, N, W = _B, _N, _W
    c2 = classic_mask.reshape(B * 512, W)
    g2 = ground_truth_mask.reshape(B * 512, W)
    s2 = snake_mask.reshape(B * 512, W)
    rows = (B * 512) // (2 * _STEPS)

    part = pl.pallas_call(
        _mask_kernel,
        grid=(2, _STEPS),
        in_specs=[
            pl.BlockSpec((rows, W), lambda i, j: (i * _STEPS + j, 0)),
            pl.BlockSpec((rows, W), lambda i, j: (i * _STEPS + j, 0)),
            pl.BlockSpec((rows, W), lambda i, j: (i * _STEPS + j, 0)),
        ],
        out_specs=pl.BlockSpec((1, 8, W), lambda i, j: (i, 0, 0)),
        out_shape=jax.ShapeDtypeStruct((2, 8, W), jnp.float32),
        compiler_params=pltpu.CompilerParams(
            dimension_semantics=("parallel", "arbitrary")),
    )(c2, g2, s2)

    preds = jnp.stack([snake_GT_size, snake_classic_size])
    refs = jnp.stack([ground_truth_contour, classic_contour])
    pbase = preds.transpose(0, 3, 1, 2).reshape(2 * 2 * B, N)
    marr = (refs.transpose(0, 3, 1, 2)
            .reshape(2, 2, B, 4, 128)
            .transpose(3, 0, 1, 2, 4)
            .reshape(4 * 2 * 2 * B, 128))

    out = pl.pallas_call(
        _corr_kernel,
        in_specs=[
            pl.BlockSpec((2 * 2 * B, N), lambda: (0, 0)),
            pl.BlockSpec((4 * 2 * 2 * B, 128), lambda: (0, 0)),
            pl.BlockSpec((2, 8, W), lambda: (0, 0, 0)),
        ],
        out_specs=pl.BlockSpec((1, 1), lambda: (0, 0)),
        out_shape=jax.ShapeDtypeStruct((1, 1), jnp.float32),
        scratch_shapes=[
            pltpu.VMEM((4 * 2 * 2 * B, N), jnp.float32),
            pltpu.VMEM((2 * 2 * B, N), jnp.float32),
            pltpu.VMEM((4 * 2 * 2 * B, 128), jnp.float32),
        ],
    )(pbase, marr, part)
    return out[0, 0]


# speedup vs baseline: 37.7286x; 1.1189x over previous
import jax
import jax.numpy as jnp
from jax.experimental import pallas as pl
from jax.experimental.pallas import tpu as pltpu

_GAMMA = 0.5
_SMOOTH = 1.0
_B = 64
_N = 512
_W = 512
_STEPS = 16


def _mask_kernel(c_ref, g_ref, s_ref, out_ref):
    j = pl.program_id(1)
    c = c_ref[...]
    g = g_ref[...]
    s = s_ref[...]
    sc = jnp.sum(c, axis=0, keepdims=True)
    sg = jnp.sum(g, axis=0, keepdims=True)
    ss = jnp.sum(s, axis=0, keepdims=True)
    scg = jnp.sum(c * g, axis=0, keepdims=True)
    scs = jnp.sum(c * s, axis=0, keepdims=True)
    block = jnp.concatenate(
        [sc, sg, ss, scg, scs, jnp.zeros((3, _W), jnp.float32)], axis=0)[None]

    @pl.when(j == 0)
    def _():
        out_ref[...] = block

    @pl.when(j != 0)
    def _():
        out_ref[...] = out_ref[...] + block


def _corr_kernel(pbase_ref, marr_ref, part_ref, out_ref, pfull_ref, acc_ref,
                 mcur_ref):
    p0 = pbase_ref[...]
    pfull_ref[0:256, :] = p0
    pfull_ref[256:512, :] = pltpu.roll(p0, 384, axis=1)
    pfull_ref[512:768, :] = pltpu.roll(p0, 256, axis=1)
    pfull_ref[768:1024, :] = pltpu.roll(p0, 128, axis=1)
    mcur_ref[...] = pltpu.roll(marr_ref[...], 1, axis=1)
    acc_ref[...] = jnp.zeros((256, _N), jnp.float32)

    def body(it, carry):
        m = mcur_ref[:, 0:1]
        t = (m * pfull_ref[...]).reshape(4, 256, _N)
        v = t[0] + t[1] + t[2] + t[3]
        acc_ref[...] = pltpu.roll(acc_ref[...], _N - 1, axis=1) + v
        mcur_ref[...] = pltpu.roll(mcur_ref[...], 1, axis=1)
        return carry

    jax.lax.fori_loop(0, 128, body, 0)

    acc = acc_ref[...]
    corr_q0 = acc[0:64, :] + acc[64:128, :]
    corr_q1 = acc[128:192, :] + acc[192:256, :]
    cmax0 = jnp.max(corr_q0, axis=1, keepdims=True)
    cmax1 = jnp.max(corr_q1, axis=1, keepdims=True)

    prow = jnp.sum(p0 * p0, axis=1, keepdims=True)
    ap0 = prow[0:64] + prow[64:128]
    ap1 = prow[128:192] + prow[192:256]
    mar = marr_ref[...]
    mm = jnp.sum(mar * mar, axis=1, keepdims=True)
    mu = mm[0:256] + mm[256:512] + mm[512:768] + mm[768:1024]
    ar0 = mu[0:64] + mu[64:128]
    ar1 = mu[128:192] + mu[192:256]

    inv2n = 1.0 / (2.0 * _N)
    mse0 = (ap0 + ar0 - 2.0 * cmax0) * inv2n
    mse1 = (ap1 + ar1 - 2.0 * cmax1) * inv2n
    seg = jnp.sum(mse0) * (1.0 / _B)
    cons = jnp.sum(mse1) * (1.0 / _B)

    p5 = part_ref[0] + part_ref[1]
    s_c = jnp.sum(p5[0:1, :])
    s_g = jnp.sum(p5[1:2, :])
    s_s = jnp.sum(p5[2:3, :])
    s_cg = jnp.sum(p5[3:4, :])
    s_cs = jnp.sum(p5[4:5, :])
    dice1 = 1.0 - (2.0 * s_cg + _SMOOTH) / (s_c + s_g + _SMOOTH)
    dice2 = 1.0 - (2.0 * s_cs + _SMOOTH) / (s_c + s_s + _SMOOTH)
    loss = (1.0 - _GAMMA) * (dice1 + seg) + _GAMMA * (dice2 + cons)
    out_ref[...] = jnp.reshape(loss, (1, 1))


def kernel(ground_truth_mask, ground_truth_contour, snake_GT_size,
           snake_classic_size, snake_mask, classic_contour, classic_mask):
    B, N, W = _B, _N, _W
    c2 = classic_mask.reshape(B * 512, W)
    g2 = ground_truth_mask.reshape(B * 512, W)
    s2 = snake_mask.reshape(B * 512, W)
    rows = (B * 512) // (2 * _STEPS)

    part = pl.pallas_call(
        _mask_kernel,
        grid=(2, _STEPS),
        in_specs=[
            pl.BlockSpec((rows, W), lambda i, j: (i * _STEPS + j, 0)),
            pl.BlockSpec((rows, W), lambda i, j: (i * _STEPS + j, 0)),
            pl.BlockSpec((rows, W), lambda i, j: (i * _STEPS + j, 0)),
        ],
        out_specs=pl.BlockSpec((1, 8, W), lambda i, j: (i, 0, 0)),
        out_shape=jax.ShapeDtypeStruct((2, 8, W), jnp.float32),
        compiler_params=pltpu.CompilerParams(
            dimension_semantics=("parallel", "arbitrary")),
    )(c2, g2, s2)

    preds = jnp.stack([snake_GT_size, snake_classic_size])
    refs = jnp.stack([ground_truth_contour, classic_contour])
    pbase = preds.transpose(0, 3, 1, 2).reshape(2 * 2 * B, N)
    marr = (refs.transpose(0, 3, 1, 2)
            .reshape(2, 2, B, 4, 128)
            .transpose(3, 0, 1, 2, 4)
            .reshape(4 * 2 * 2 * B, 128))

    out = pl.pallas_call(
        _corr_kernel,
        in_specs=[
            pl.BlockSpec((2 * 2 * B, N), lambda: (0, 0)),
            pl.BlockSpec((4 * 2 * 2 * B, 128), lambda: (0, 0)),
            pl.BlockSpec((2, 8, W), lambda: (0, 0, 0)),
        ],
        out_specs=pl.BlockSpec((1, 1), lambda: (0, 0)),
        out_shape=jax.ShapeDtypeStruct((1, 1), jnp.float32),
        scratch_shapes=[
            pltpu.VMEM((4 * 2 * 2 * B, N), jnp.float32),
            pltpu.VMEM((2 * 2 * B, N), jnp.float32),
            pltpu.VMEM((4 * 2 * 2 * B, 128), jnp.float32),
        ],
    )(pbase, marr, part)
    return out[0, 0]


# speedup vs baseline: 38.1257x; 1.0105x over previous
import jax
import jax.numpy as jnp
from jax.experimental import pallas as pl
from jax.experimental.pallas import tpu as pltpu

_GAMMA = 0.5
_SMOOTH = 1.0
_B = 64
_N = 512
_W = 512
_STEPS = 8


def _mask_kernel(c_ref, g_ref, s_ref, out_ref):
    j = pl.program_id(1)
    c = c_ref[...]
    g = g_ref[...]
    s = s_ref[...]
    sc = jnp.sum(c, axis=0, keepdims=True)
    sg = jnp.sum(g, axis=0, keepdims=True)
    ss = jnp.sum(s, axis=0, keepdims=True)
    scg = jnp.sum(c * g, axis=0, keepdims=True)
    scs = jnp.sum(c * s, axis=0, keepdims=True)
    block = jnp.concatenate(
        [sc, sg, ss, scg, scs, jnp.zeros((3, _W), jnp.float32)], axis=0)[None]

    @pl.when(j == 0)
    def _():
        out_ref[...] = block

    @pl.when(j != 0)
    def _():
        out_ref[...] = out_ref[...] + block


def _corr_kernel(pbase_ref, marr_ref, part_ref, out_ref, pfull_ref, acc_ref,
                 mcur_ref):
    p0 = pbase_ref[...]
    pfull_ref[0:256, :] = p0
    pfull_ref[256:512, :] = pltpu.roll(p0, 384, axis=1)
    pfull_ref[512:768, :] = pltpu.roll(p0, 256, axis=1)
    pfull_ref[768:1024, :] = pltpu.roll(p0, 128, axis=1)
    mcur_ref[...] = pltpu.roll(marr_ref[...], 1, axis=1)
    acc_ref[...] = jnp.zeros((256, _N), jnp.float32)

    def body(it, carry):
        m = mcur_ref[:, 0:1]
        t = (m * pfull_ref[...]).reshape(4, 256, _N)
        v = t[0] + t[1] + t[2] + t[3]
        acc_ref[...] = pltpu.roll(acc_ref[...], _N - 1, axis=1) + v
        mcur_ref[...] = pltpu.roll(mcur_ref[...], 1, axis=1)
        return carry

    jax.lax.fori_loop(0, 128, body, 0)

    acc = acc_ref[...]
    corr_q0 = acc[0:64, :] + acc[64:128, :]
    corr_q1 = acc[128:192, :] + acc[192:256, :]
    cmax0 = jnp.max(corr_q0, axis=1, keepdims=True)
    cmax1 = jnp.max(corr_q1, axis=1, keepdims=True)

    prow = jnp.sum(p0 * p0, axis=1, keepdims=True)
    ap0 = prow[0:64] + prow[64:128]
    ap1 = prow[128:192] + prow[192:256]
    mar = marr_ref[...]
    mm = jnp.sum(mar * mar, axis=1, keepdims=True)
    mu = mm[0:256] + mm[256:512] + mm[512:768] + mm[768:1024]
    ar0 = mu[0:64] + mu[64:128]
    ar1 = mu[128:192] + mu[192:256]

    inv2n = 1.0 / (2.0 * _N)
    mse0 = (ap0 + ar0 - 2.0 * cmax0) * inv2n
    mse1 = (ap1 + ar1 - 2.0 * cmax1) * inv2n
    seg = jnp.sum(mse0) * (1.0 / _B)
    cons = jnp.sum(mse1) * (1.0 / _B)

    p5 = part_ref[0] + part_ref[1]
    s_c = jnp.sum(p5[0:1, :])
    s_g = jnp.sum(p5[1:2, :])
    s_s = jnp.sum(p5[2:3, :])
    s_cg = jnp.sum(p5[3:4, :])
    s_cs = jnp.sum(p5[4:5, :])
    dice1 = 1.0 - (2.0 * s_cg + _SMOOTH) / (s_c + s_g + _SMOOTH)
    dice2 = 1.0 - (2.0 * s_cs + _SMOOTH) / (s_c + s_s + _SMOOTH)
    loss = (1.0 - _GAMMA) * (dice1 + seg) + _GAMMA * (dice2 + cons)
    out_ref[...] = jnp.reshape(loss, (1, 1))


def kernel(ground_truth_mask, ground_truth_contour, snake_GT_size,
           snake_classic_size, snake_mask, classic_contour, classic_mask):
    B, N, W = _B, _N, _W
    c2 = classic_mask.reshape(B * 512, W)
    g2 = ground_truth_mask.reshape(B * 512, W)
    s2 = snake_mask.reshape(B * 512, W)
    rows = (B * 512) // (2 * _STEPS)

    part = pl.pallas_call(
        _mask_kernel,
        grid=(2, _STEPS),
        in_specs=[
            pl.BlockSpec((rows, W), lambda i, j: (i * _STEPS + j, 0)),
            pl.BlockSpec((rows, W), lambda i, j: (i * _STEPS + j, 0)),
            pl.BlockSpec((rows, W), lambda i, j: (i * _STEPS + j, 0)),
        ],
        out_specs=pl.BlockSpec((1, 8, W), lambda i, j: (i, 0, 0)),
        out_shape=jax.ShapeDtypeStruct((2, 8, W), jnp.float32),
        compiler_params=pltpu.CompilerParams(
            dimension_semantics=("parallel", "arbitrary")),
    )(c2, g2, s2)

    preds = jnp.stack([snake_GT_size, snake_classic_size])
    refs = jnp.stack([ground_truth_contour, classic_contour])
    pbase = preds.transpose(0, 3, 1, 2).reshape(2 * 2 * B, N)
    marr = (refs.transpose(0, 3, 1, 2)
            .reshape(2, 2, B, 4, 128)
            .transpose(3, 0, 1, 2, 4)
            .reshape(4 * 2 * 2 * B, 128))

    out = pl.pallas_call(
        _corr_kernel,
        in_specs=[
            pl.BlockSpec((2 * 2 * B, N), lambda: (0, 0)),
            pl.BlockSpec((4 * 2 * 2 * B, 128), lambda: (0, 0)),
            pl.BlockSpec((2, 8, W), lambda: (0, 0, 0)),
        ],
        out_specs=pl.BlockSpec((1, 1), lambda: (0, 0)),
        out_shape=jax.ShapeDtypeStruct((1, 1), jnp.float32),
        scratch_shapes=[
            pltpu.VMEM((4 * 2 * 2 * B, N), jnp.float32),
            pltpu.VMEM((2 * 2 * B, N), jnp.float32),
            pltpu.VMEM((4 * 2 * 2 * B, 128), jnp.float32),
        ],
    )(pbase, marr, part)
    return out[0, 0]


# speedup vs baseline: 57.7111x; 1.5137x over previous
import jax
import jax.numpy as jnp
from jax.experimental import pallas as pl
from jax.experimental.pallas import tpu as pltpu

_GAMMA = 0.5
_SMOOTH = 1.0
_B = 64
_N = 512
_W = 512
_STEPS = 8


def _mask_kernel(c_ref, g_ref, s_ref, out_ref):
    j = pl.program_id(1)
    c = c_ref[...]
    g = g_ref[...]
    s = s_ref[...]
    sc = jnp.sum(c, axis=0, keepdims=True)
    sg = jnp.sum(g, axis=0, keepdims=True)
    ss = jnp.sum(s, axis=0, keepdims=True)
    scg = jnp.sum(c * g, axis=0, keepdims=True)
    scs = jnp.sum(c * s, axis=0, keepdims=True)
    block = jnp.concatenate(
        [sc, sg, ss, scg, scs, jnp.zeros((3, _W), jnp.float32)], axis=0)[None]

    @pl.when(j == 0)
    def _():
        out_ref[...] = block

    @pl.when(j != 0)
    def _():
        out_ref[...] = out_ref[...] + block


def _corr_kernel(ptr_ref, mtr_ref, out_ref, pfull_ref, acc_ref):
    pbase = ptr_ref[...]
    pfull_ref[:, 0:128] = pbase
    pfull_ref[:, 128:256] = jnp.concatenate([pbase[128:], pbase[:128]], axis=0)
    pfull_ref[:, 256:384] = jnp.concatenate([pbase[256:], pbase[:256]], axis=0)
    pfull_ref[:, 384:512] = jnp.concatenate([pbase[384:], pbase[:384]], axis=0)
    acc_ref[...] = jnp.zeros((_N, 128), jnp.float32)

    def body(it, carry):
        w = 127 - it
        m = mtr_ref[pl.ds(w, 1), 0, :]
        t = m * pfull_ref[...]
        v = (t[:, 0:128] + t[:, 128:256]
             + t[:, 256:384] + t[:, 384:512])
        acc_ref[...] = pltpu.roll(acc_ref[...], _N - 1, axis=0) + v
        return carry

    jax.lax.fori_loop(0, 128, body, 0)

    acc = acc_ref[...]
    corr = acc[:, 0:64] + acc[:, 64:128]
    cmax = jnp.max(corr, axis=0, keepdims=True)

    pq = jnp.sum(pbase * pbase, axis=0, keepdims=True)
    ap = pq[:, 0:64] + pq[:, 64:128]
    msq = mtr_ref[...][:, 0, :]
    mq = jnp.sum(msq * msq, axis=0, keepdims=True)
    mu = (mq[:, 0:128] + mq[:, 128:256]
          + mq[:, 256:384] + mq[:, 384:512])
    ar = mu[:, 0:64] + mu[:, 64:128]

    mse = (ap + ar - 2.0 * cmax) * (1.0 / (2.0 * _N))
    out_ref[...] = jnp.concatenate(
        [mse, jnp.zeros((1, 64), jnp.float32)], axis=1)


def _final_kernel(part_ref, cpart_ref, out_ref):
    p5 = part_ref[0] + part_ref[1]
    s_c = jnp.sum(p5[0:1, :])
    s_g = jnp.sum(p5[1:2, :])
    s_s = jnp.sum(p5[2:3, :])
    s_cg = jnp.sum(p5[3:4, :])
    s_cs = jnp.sum(p5[4:5, :])
    seg = jnp.sum(cpart_ref[0][:, 0:64]) * (1.0 / _B)
    cons = jnp.sum(cpart_ref[1][:, 0:64]) * (1.0 / _B)
    dice1 = 1.0 - (2.0 * s_cg + _SMOOTH) / (s_c + s_g + _SMOOTH)
    dice2 = 1.0 - (2.0 * s_cs + _SMOOTH) / (s_c + s_s + _SMOOTH)
    loss = (1.0 - _GAMMA) * (dice1 + seg) + _GAMMA * (dice2 + cons)
    out_ref[...] = jnp.reshape(loss, (1, 1))


def kernel(ground_truth_mask, ground_truth_contour, snake_GT_size,
           snake_classic_size, snake_mask, classic_contour, classic_mask):
    B, N, W = _B, _N, _W
    c2 = classic_mask.reshape(B * 512, W)
    g2 = ground_truth_mask.reshape(B * 512, W)
    s2 = snake_mask.reshape(B * 512, W)
    rows = (B * 512) // (2 * _STEPS)

    part = pl.pallas_call(
        _mask_kernel,
        grid=(2, _STEPS),
        in_specs=[
            pl.BlockSpec((rows, W), lambda i, j: (i * _STEPS + j, 0)),
            pl.BlockSpec((rows, W), lambda i, j: (i * _STEPS + j, 0)),
            pl.BlockSpec((rows, W), lambda i, j: (i * _STEPS + j, 0)),
        ],
        out_specs=pl.BlockSpec((1, 8, W), lambda i, j: (i, 0, 0)),
        out_shape=jax.ShapeDtypeStruct((2, 8, W), jnp.float32),
        compiler_params=pltpu.CompilerParams(
            dimension_semantics=("parallel", "arbitrary")),
    )(c2, g2, s2)

    preds = jnp.stack([snake_GT_size, snake_classic_size])
    refs = jnp.stack([ground_truth_contour, classic_contour])
    ptr = preds.transpose(0, 2, 3, 1).reshape(2, N, 2 * B)
    mtr = (refs.transpose(0, 2, 3, 1)
           .reshape(2, 4, 128, 2, B)
           .transpose(0, 2, 1, 3, 4)
           .reshape(2, 128, 1, 4 * 2 * B))

    cpart = pl.pallas_call(
        _corr_kernel,
        grid=(2,),
        in_specs=[
            pl.BlockSpec((None, N, 2 * B), lambda i: (i, 0, 0)),
            pl.BlockSpec((None, 128, 1, 4 * 2 * B), lambda i: (i, 0, 0, 0)),
        ],
        out_specs=pl.BlockSpec((None, 1, 128), lambda i: (i, 0, 0)),
        out_shape=jax.ShapeDtypeStruct((2, 1, 128), jnp.float32),
        scratch_shapes=[
            pltpu.VMEM((N, 4 * 2 * B), jnp.float32),
            pltpu.VMEM((N, 2 * B), jnp.float32),
        ],
        compiler_params=pltpu.CompilerParams(
            dimension_semantics=("parallel",)),
    )(ptr, mtr)

    out = pl.pallas_call(
        _final_kernel,
        in_specs=[
            pl.BlockSpec((2, 8, W), lambda: (0, 0, 0)),
            pl.BlockSpec((2, 1, 128), lambda: (0, 0, 0)),
        ],
        out_specs=pl.BlockSpec((1, 1), lambda: (0, 0)),
        out_shape=jax.ShapeDtypeStruct((1, 1), jnp.float32),
    )(part, cpart)
    return out[0, 0]


# speedup vs baseline: 71.9509x; 1.2467x over previous
import jax
import jax.numpy as jnp
from jax.experimental import pallas as pl
from jax.experimental.pallas import tpu as pltpu

_GAMMA = 0.5
_SMOOTH = 1.0
_B = 64
_N = 512
_W = 512
_STEPS = 8
_ITERS = 128 // _STEPS


def _fused_kernel(c_ref, g_ref, s_ref, ptr_ref, mtr_ref,
                  part_ref, cpart_ref, pfull_ref, acc_ref):
    j = pl.program_id(1)

    @pl.when(j == 0)
    def _():
        pbase = ptr_ref[...]
        pfull_ref[:, 0:128] = pbase
        pfull_ref[:, 128:256] = jnp.concatenate(
            [pbase[128:], pbase[:128]], axis=0)
        pfull_ref[:, 256:384] = jnp.concatenate(
            [pbase[256:], pbase[:256]], axis=0)
        pfull_ref[:, 384:512] = jnp.concatenate(
            [pbase[384:], pbase[:384]], axis=0)
        acc_ref[...] = jnp.zeros((_N, 128), jnp.float32)

    c = c_ref[...]
    g = g_ref[...]
    s = s_ref[...]
    sc = jnp.sum(c, axis=0, keepdims=True)
    sg = jnp.sum(g, axis=0, keepdims=True)
    ss = jnp.sum(s, axis=0, keepdims=True)
    scg = jnp.sum(c * g, axis=0, keepdims=True)
    scs = jnp.sum(c * s, axis=0, keepdims=True)
    block = jnp.concatenate(
        [sc, sg, ss, scg, scs, jnp.zeros((3, _W), jnp.float32)], axis=0)[None]

    @pl.when(j == 0)
    def _():
        part_ref[...] = block

    @pl.when(j != 0)
    def _():
        part_ref[...] = part_ref[...] + block

    def body(it, carry):
        w = 127 - (j * _ITERS + it)
        m = mtr_ref[pl.ds(w, 1), 0, :]
        t = m * pfull_ref[...]
        v = (t[:, 0:128] + t[:, 128:256]
             + t[:, 256:384] + t[:, 384:512])
        acc_ref[...] = pltpu.roll(acc_ref[...], _N - 1, axis=0) + v
        return carry

    jax.lax.fori_loop(0, _ITERS, body, 0)

    @pl.when(j == _STEPS - 1)
    def _():
        acc = acc_ref[...]
        corr = acc[:, 0:64] + acc[:, 64:128]
        cmax = jnp.max(corr, axis=0, keepdims=True)

        pbase = ptr_ref[...]
        pq = jnp.sum(pbase * pbase, axis=0, keepdims=True)
        ap = pq[:, 0:64] + pq[:, 64:128]
        msq = mtr_ref[...][:, 0, :]
        mq = jnp.sum(msq * msq, axis=0, keepdims=True)
        mu = (mq[:, 0:128] + mq[:, 128:256]
              + mq[:, 256:384] + mq[:, 384:512])
        ar = mu[:, 0:64] + mu[:, 64:128]

        mse = (ap + ar - 2.0 * cmax) * (1.0 / (2.0 * _N))
        cpart_ref[...] = jnp.concatenate(
            [mse, jnp.zeros((1, 64), jnp.float32)], axis=1)


def _final_kernel(part_ref, cpart_ref, out_ref):
    p5 = part_ref[0] + part_ref[1]
    s_c = jnp.sum(p5[0:1, :])
    s_g = jnp.sum(p5[1:2, :])
    s_s = jnp.sum(p5[2:3, :])
    s_cg = jnp.sum(p5[3:4, :])
    s_cs = jnp.sum(p5[4:5, :])
    seg = jnp.sum(cpart_ref[0][:, 0:64]) * (1.0 / _B)
    cons = jnp.sum(cpart_ref[1][:, 0:64]) * (1.0 / _B)
    dice1 = 1.0 - (2.0 * s_cg + _SMOOTH) / (s_c + s_g + _SMOOTH)
    dice2 = 1.0 - (2.0 * s_cs + _SMOOTH) / (s_c + s_s + _SMOOTH)
    loss = (1.0 - _GAMMA) * (dice1 + seg) + _GAMMA * (dice2 + cons)
    out_ref[...] = jnp.reshape(loss, (1, 1))


def kernel(ground_truth_mask, ground_truth_contour, snake_GT_size,
           snake_classic_size, snake_mask, classic_contour, classic_mask):
    B, N, W = _B, _N, _W
    c2 = classic_mask.reshape(B * 512, W)
    g2 = ground_truth_mask.reshape(B * 512, W)
    s2 = snake_mask.reshape(B * 512, W)
    rows = (B * 512) // (2 * _STEPS)

    preds = jnp.stack([snake_GT_size, snake_classic_size])
    refs = jnp.stack([ground_truth_contour, classic_contour])
    ptr = preds.transpose(0, 2, 3, 1).reshape(2, N, 2 * B)
    mtr = (refs.transpose(0, 2, 3, 1)
           .reshape(2, 4, 128, 2, B)
           .transpose(0, 2, 1, 3, 4)
           .reshape(2, 128, 1, 4 * 2 * B))

    part, cpart = pl.pallas_call(
        _fused_kernel,
        grid=(2, _STEPS),
        in_specs=[
            pl.BlockSpec((rows, W), lambda i, j: (i * _STEPS + j, 0)),
            pl.BlockSpec((rows, W), lambda i, j: (i * _STEPS + j, 0)),
            pl.BlockSpec((rows, W), lambda i, j: (i * _STEPS + j, 0)),
            pl.BlockSpec((None, N, 2 * B), lambda i, j: (i, 0, 0)),
            pl.BlockSpec((None, 128, 1, 4 * 2 * B), lambda i, j: (i, 0, 0, 0)),
        ],
        out_specs=[
            pl.BlockSpec((1, 8, W), lambda i, j: (i, 0, 0)),
            pl.BlockSpec((None, 1, 128), lambda i, j: (i, 0, 0)),
        ],
        out_shape=[
            jax.ShapeDtypeStruct((2, 8, W), jnp.float32),
            jax.ShapeDtypeStruct((2, 1, 128), jnp.float32),
        ],
        scratch_shapes=[
            pltpu.VMEM((N, 4 * 2 * B), jnp.float32),
            pltpu.VMEM((N, 2 * B), jnp.float32),
        ],
        compiler_params=pltpu.CompilerParams(
            dimension_semantics=("parallel", "arbitrary")),
    )(c2, g2, s2, ptr, mtr)

    out = pl.pallas_call(
        _final_kernel,
        in_specs=[
            pl.BlockSpec((2, 8, W), lambda: (0, 0, 0)),
            pl.BlockSpec((2, 1, 128), lambda: (0, 0, 0)),
        ],
        out_specs=pl.BlockSpec((1, 1), lambda: (0, 0)),
        out_shape=jax.ShapeDtypeStruct((1, 1), jnp.float32),
    )(part, cpart)
    return out[0, 0]


# speedup vs baseline: 72.5526x; 1.0084x over previous
import jax
import jax.numpy as jnp
from jax.experimental import pallas as pl
from jax.experimental.pallas import tpu as pltpu

_GAMMA = 0.5
_SMOOTH = 1.0
_B = 64
_N = 512
_W = 512
_STEPS = 4
_ITERS = 128 // _STEPS


def _fused_kernel(c_ref, g_ref, s_ref, ptr_ref, mtr_ref,
                  part_ref, cpart_ref, pfull_ref, acc_ref):
    j = pl.program_id(1)

    @pl.when(j == 0)
    def _():
        pbase = ptr_ref[...]
        pfull_ref[:, 0:128] = pbase
        pfull_ref[:, 128:256] = jnp.concatenate(
            [pbase[128:], pbase[:128]], axis=0)
        pfull_ref[:, 256:384] = jnp.concatenate(
            [pbase[256:], pbase[:256]], axis=0)
        pfull_ref[:, 384:512] = jnp.concatenate(
            [pbase[384:], pbase[:384]], axis=0)
        acc_ref[...] = jnp.zeros((_N, 128), jnp.float32)

    c = c_ref[...]
    g = g_ref[...]
    s = s_ref[...]
    sc = jnp.sum(c, axis=0, keepdims=True)
    sg = jnp.sum(g, axis=0, keepdims=True)
    ss = jnp.sum(s, axis=0, keepdims=True)
    scg = jnp.sum(c * g, axis=0, keepdims=True)
    scs = jnp.sum(c * s, axis=0, keepdims=True)
    block = jnp.concatenate(
        [sc, sg, ss, scg, scs, jnp.zeros((3, _W), jnp.float32)], axis=0)[None]

    @pl.when(j == 0)
    def _():
        part_ref[...] = block

    @pl.when(j != 0)
    def _():
        part_ref[...] = part_ref[...] + block

    def body(it, carry):
        w = 127 - (j * _ITERS + it)
        m = mtr_ref[pl.ds(w, 1), 0, :]
        t = m * pfull_ref[...]
        v = (t[:, 0:128] + t[:, 128:256]
             + t[:, 256:384] + t[:, 384:512])
        acc_ref[...] = pltpu.roll(acc_ref[...], _N - 1, axis=0) + v
        return carry

    jax.lax.fori_loop(0, _ITERS, body, 0)

    @pl.when(j == _STEPS - 1)
    def _():
        acc = acc_ref[...]
        corr = acc[:, 0:64] + acc[:, 64:128]
        cmax = jnp.max(corr, axis=0, keepdims=True)

        pbase = ptr_ref[...]
        pq = jnp.sum(pbase * pbase, axis=0, keepdims=True)
        ap = pq[:, 0:64] + pq[:, 64:128]
        msq = mtr_ref[...][:, 0, :]
        mq = jnp.sum(msq * msq, axis=0, keepdims=True)
        mu = (mq[:, 0:128] + mq[:, 128:256]
              + mq[:, 256:384] + mq[:, 384:512])
        ar = mu[:, 0:64] + mu[:, 64:128]

        mse = (ap + ar - 2.0 * cmax) * (1.0 / (2.0 * _N))
        cpart_ref[...] = jnp.concatenate(
            [mse, jnp.zeros((1, 64), jnp.float32)], axis=1)


def _final_kernel(part_ref, cpart_ref, out_ref):
    p5 = part_ref[0] + part_ref[1]
    s_c = jnp.sum(p5[0:1, :])
    s_g = jnp.sum(p5[1:2, :])
    s_s = jnp.sum(p5[2:3, :])
    s_cg = jnp.sum(p5[3:4, :])
    s_cs = jnp.sum(p5[4:5, :])
    seg = jnp.sum(cpart_ref[0][:, 0:64]) * (1.0 / _B)
    cons = jnp.sum(cpart_ref[1][:, 0:64]) * (1.0 / _B)
    dice1 = 1.0 - (2.0 * s_cg + _SMOOTH) / (s_c + s_g + _SMOOTH)
    dice2 = 1.0 - (2.0 * s_cs + _SMOOTH) / (s_c + s_s + _SMOOTH)
    loss = (1.0 - _GAMMA) * (dice1 + seg) + _GAMMA * (dice2 + cons)
    out_ref[...] = jnp.reshape(loss, (1, 1))


def kernel(ground_truth_mask, ground_truth_contour, snake_GT_size,
           snake_classic_size, snake_mask, classic_contour, classic_mask):
    B, N, W = _B, _N, _W
    c2 = classic_mask.reshape(B * 512, W)
    g2 = ground_truth_mask.reshape(B * 512, W)
    s2 = snake_mask.reshape(B * 512, W)
    rows = (B * 512) // (2 * _STEPS)

    preds = jnp.stack([snake_GT_size, snake_classic_size])
    refs = jnp.stack([ground_truth_contour, classic_contour])
    ptr = preds.transpose(0, 2, 3, 1).reshape(2, N, 2 * B)
    mtr = (refs.transpose(0, 2, 3, 1)
           .reshape(2, 4, 128, 2, B)
           .transpose(0, 2, 1, 3, 4)
           .reshape(2, 128, 1, 4 * 2 * B))

    part, cpart = pl.pallas_call(
        _fused_kernel,
        grid=(2, _STEPS),
        in_specs=[
            pl.BlockSpec((rows, W), lambda i, j: (i * _STEPS + j, 0)),
            pl.BlockSpec((rows, W), lambda i, j: (i * _STEPS + j, 0)),
            pl.BlockSpec((rows, W), lambda i, j: (i * _STEPS + j, 0)),
            pl.BlockSpec((None, N, 2 * B), lambda i, j: (i, 0, 0)),
            pl.BlockSpec((None, 128, 1, 4 * 2 * B), lambda i, j: (i, 0, 0, 0)),
        ],
        out_specs=[
            pl.BlockSpec((1, 8, W), lambda i, j: (i, 0, 0)),
            pl.BlockSpec((None, 1, 128), lambda i, j: (i, 0, 0)),
        ],
        out_shape=[
            jax.ShapeDtypeStruct((2, 8, W), jnp.float32),
            jax.ShapeDtypeStruct((2, 1, 128), jnp.float32),
        ],
        scratch_shapes=[
            pltpu.VMEM((N, 4 * 2 * B), jnp.float32),
            pltpu.VMEM((N, 2 * B), jnp.float32),
        ],
        compiler_params=pltpu.CompilerParams(
            dimension_semantics=("parallel", "arbitrary")),
    )(c2, g2, s2, ptr, mtr)

    out = pl.pallas_call(
        _final_kernel,
        in_specs=[
            pl.BlockSpec((2, 8, W), lambda: (0, 0, 0)),
            pl.BlockSpec((2, 1, 128), lambda: (0, 0, 0)),
        ],
        out_specs=pl.BlockSpec((1, 1), lambda: (0, 0)),
        out_shape=jax.ShapeDtypeStruct((1, 1), jnp.float32),
    )(part, cpart)
    return out[0, 0]
